# baseline (speedup 1.0000x reference)
import jax
import jax.numpy as jnp
from jax.experimental import pallas as pl

_BATCH = 32
_SEQ_LEN = 2048
_MODEL_DIM = 1024
_SPAN_LEN = 10
_MAX_MASK_PROB = 0.65
_MIN_NUM_SPANS = 2

_ROWS_PER_BLOCK = 512


def _temporal_mask():
    num_spans = max(_MIN_NUM_SPANS, int(_MAX_MASK_PROB * _SEQ_LEN / _SPAN_LEN))
    mask_key = jax.random.fold_in(jax.random.key(0), 12345)
    starts = jax.random.randint(mask_key, (_BATCH, num_spans), 0, _SEQ_LEN - _SPAN_LEN)
    offsets = jnp.arange(_SPAN_LEN)
    idx = starts[:, :, None] + offsets[None, None, :]
    row_idx = jnp.broadcast_to(jnp.arange(_BATCH)[:, None, None], idx.shape)
    mask = jnp.zeros((_BATCH, _SEQ_LEN), dtype=bool)
    mask = mask.at[row_idx, idx].set(True)
    return mask


def _overwrite_body(mask_ref, embed_ref, seqs_ref, out_ref):
    m = mask_ref[:, :] > 0
    out_ref[:, :] = jnp.where(m, embed_ref[:, :], seqs_ref[:, :])


def kernel(seqs, temporal_mask_embed):
    mask = _temporal_mask()
    n_rows = _BATCH * _SEQ_LEN
    seqs2d = seqs.reshape(n_rows, _MODEL_DIM)
    maskf = mask.reshape(n_rows, 1).astype(jnp.float32)
    embed2d = temporal_mask_embed.reshape(1, _MODEL_DIM)

    grid = (n_rows // _ROWS_PER_BLOCK,)
    out2d = pl.pallas_call(
        _overwrite_body,
        grid=grid,
        in_specs=[
            pl.BlockSpec((_ROWS_PER_BLOCK, 1), lambda i: (i, 0)),
            pl.BlockSpec((1, _MODEL_DIM), lambda i: (0, 0)),
            pl.BlockSpec((_ROWS_PER_BLOCK, _MODEL_DIM), lambda i: (i, 0)),
        ],
        out_specs=pl.BlockSpec((_ROWS_PER_BLOCK, _MODEL_DIM), lambda i: (i, 0)),
        out_shape=jax.ShapeDtypeStruct((n_rows, _MODEL_DIM), seqs.dtype),
    )(maskf, embed2d, seqs2d)

    return out2d.reshape(_BATCH, _SEQ_LEN, _MODEL_DIM), mask
